# baseline (speedup 1.0000x reference)
import functools

import jax
import jax.numpy as jnp
from jax import lax
from jax.experimental import pallas as pl
from jax.experimental.pallas import tpu as pltpu
from jax.experimental.pallas import tpu_sc as plsc

MAX_SLOT = 20
TIME_STEPS = 20
NUM_SLOTS = 18
NUM_EMB = 1002
EMB_DIM = 32
BATCH = 1024

NUM_WORKERS = 32
NUM_FACES = TIME_STEPS * NUM_SLOTS
LANES = 16
B_CHUNKS = BATCH // LANES
HALF = EMB_DIM // 2


NUM_UNITS = NUM_FACES * 2


def _sc_body(tab_hbm, raw_hbm, out_hbm,
             ids_a, ids_b, face_a, face_b, stage_a, stage_b,
             sem_ia, sem_ib, sem_oa, sem_ob):
    wid = lax.axis_index("s") * 2 + lax.axis_index("c")
    u_lo = (wid * NUM_UNITS) // NUM_WORKERS
    u_hi = ((wid + 1) * NUM_UNITS) // NUM_WORKERS
    idss, faces = (ids_a, ids_b), (face_a, face_b)
    sem_i = (sem_ia, sem_ib)
    stages, sem_o = (stage_a, stage_b), (sem_oa, sem_ob)

    def unit_tjsh(u):
        f = u // 2
        hh = u % 2
        j = f % NUM_SLOTS
        slot = j + 1 + (j >= 4).astype(jnp.int32)
        return f // NUM_SLOTS, j, slot, hh

    def in_copies(u, p):
        t, j, slot, hh = unit_tjsh(u)
        row = pl.multiple_of(HALF * hh, HALF)
        return (pltpu.make_async_copy(raw_hbm.at[t, slot], idss[p], sem_i[p]),
                pltpu.make_async_copy(
                    tab_hbm.at[t, j, pl.ds(row, HALF), :], faces[p],
                    sem_i[p]))

    def fire_in(u, p):
        for cp in in_copies(u, p):
            cp.start()

    def wait_in(u, p):
        for cp in in_copies(u, p):
            cp.wait()

    def out_copy(u, p):
        t, j, _, hh = unit_tjsh(u)
        row0 = pl.multiple_of(EMB_DIM * j + HALF * hh, HALF)
        return pltpu.make_async_copy(
            stages[p], out_hbm.at[t, pl.ds(row0, HALF), :], sem_o[p])

    def unit_compute(u, r, pi, ps):
        @pl.when(u + 1 < u_hi)
        def _prefetch():
            fire_in(u + 1, 1 - pi)

        wait_in(u, pi)

        @pl.when(r >= 2)
        def _drain():
            out_copy(u, ps).wait()

        def chunk_body(bc, carry, pi=pi, ps=ps):
            ev = idss[pi][pl.ds(bc * LANES, LANES)]
            vals = [
                plsc.load_gather(
                    faces[pi], [jnp.full((LANES,), d, jnp.int32), ev])
                for d in range(HALF)
            ]
            for d in range(HALF):
                stages[ps][d, pl.ds(bc * LANES, LANES)] = vals[d]
            return carry

        lax.fori_loop(0, B_CHUNKS, chunk_body, 0)
        out_copy(u, ps).start()

    fire_in(u_lo, 0)

    def unit_body(r, carry):
        u = u_lo + r
        for k in range(2):
            @pl.when(r % 2 == k)
            def _unit(k=k):
                unit_compute(u, r, k, k)

        return carry

    lax.fori_loop(0, u_hi - u_lo, unit_body, 0)
    out_copy(u_lo, 0).wait()
    out_copy(u_lo, 1).wait()


@jax.jit
def _run(tab, raw):
    mesh = plsc.VectorSubcoreMesh(core_axis_name="c", subcore_axis_name="s")
    f = functools.partial(
        pl.kernel,
        mesh=mesh,
        compiler_params=pltpu.CompilerParams(needs_layout_passes=False),
        out_type=jax.ShapeDtypeStruct(
            (TIME_STEPS, NUM_SLOTS * EMB_DIM, BATCH), jnp.float32),
        scratch_types=[
            pltpu.VMEM((BATCH,), jnp.int32),
            pltpu.VMEM((BATCH,), jnp.int32),
            pltpu.VMEM((HALF, NUM_EMB), jnp.float32),
            pltpu.VMEM((HALF, NUM_EMB), jnp.float32),
            pltpu.VMEM((HALF, BATCH), jnp.float32),
            pltpu.VMEM((HALF, BATCH), jnp.float32),
            pltpu.SemaphoreType.DMA,
            pltpu.SemaphoreType.DMA,
            pltpu.SemaphoreType.DMA,
            pltpu.SemaphoreType.DMA,
        ],
    )(_sc_body)
    return f(tab, raw)


def kernel(sequence_bucket_inputs, tables):
    tab_t = tables.transpose(0, 1, 3, 2)
    raw_t = sequence_bucket_inputs.transpose(1, 2, 0)
    out = _run(tab_t, raw_t)
    return out.transpose(2, 0, 1)


# speedup vs baseline: 1.0075x; 1.0075x over previous
import functools

import jax
import jax.numpy as jnp
from jax import lax
from jax.experimental import pallas as pl
from jax.experimental.pallas import tpu as pltpu
from jax.experimental.pallas import tpu_sc as plsc

MAX_SLOT = 20
TIME_STEPS = 20
NUM_SLOTS = 18
NUM_EMB = 1002
EMB_DIM = 32
BATCH = 1024

NUM_WORKERS = 32
NUM_FACES = TIME_STEPS * NUM_SLOTS
LANES = 16
B_CHUNKS = BATCH // LANES
HALF = EMB_DIM // 2


NUM_UNITS = NUM_FACES * 2


def _sc_body(tab_hbm, raw_hbm, out_hbm,
             ids_a, ids_b, face_a, face_b, stage_a, stage_b,
             sem_ia, sem_ib, sem_oa, sem_ob):
    wid = lax.axis_index("s") * 2 + lax.axis_index("c")
    u_lo = (wid * NUM_UNITS) // NUM_WORKERS
    u_hi = ((wid + 1) * NUM_UNITS) // NUM_WORKERS
    idss, faces = (ids_a, ids_b), (face_a, face_b)
    sem_i = (sem_ia, sem_ib)
    stages, sem_o = (stage_a, stage_b), (sem_oa, sem_ob)

    def unit_tjsh(u):
        f = u // 2
        hh = u % 2
        j = f % NUM_SLOTS
        slot = j + 1 + (j >= 4).astype(jnp.int32)
        return f // NUM_SLOTS, j, slot, hh

    def in_copies(u, p):
        t, j, slot, hh = unit_tjsh(u)
        row = pl.multiple_of(HALF * hh, HALF)
        return (pltpu.make_async_copy(raw_hbm.at[t, slot], idss[p], sem_i[p]),
                pltpu.make_async_copy(
                    tab_hbm.at[t, j, pl.ds(row, HALF), :], faces[p],
                    sem_i[p]))

    def fire_in(u, p):
        for cp in in_copies(u, p):
            cp.start()

    def wait_in(u, p):
        for cp in in_copies(u, p):
            cp.wait()

    def out_copy(u, p):
        t, j, _, hh = unit_tjsh(u)
        row0 = pl.multiple_of(EMB_DIM * j + HALF * hh, HALF)
        return pltpu.make_async_copy(
            stages[p], out_hbm.at[t, pl.ds(row0, HALF), :], sem_o[p])

    def unit_compute(u, r, pi, ps):
        wait_in(u, pi)

        @pl.when(u + 1 < u_hi)
        def _prefetch():
            fire_in(u + 1, 1 - pi)

        @pl.when(r >= 2)
        def _drain():
            out_copy(u, ps).wait()

        def chunk_body(bc, carry, pi=pi, ps=ps):
            ev = idss[pi][pl.ds(bc * LANES, LANES)]
            vals = [
                plsc.load_gather(
                    faces[pi], [jnp.full((LANES,), d, jnp.int32), ev])
                for d in range(HALF)
            ]
            for d in range(HALF):
                stages[ps][d, pl.ds(bc * LANES, LANES)] = vals[d]
            return carry

        lax.fori_loop(0, B_CHUNKS, chunk_body, 0)
        out_copy(u, ps).start()

    fire_in(u_lo, 0)

    def unit_body(r, carry):
        u = u_lo + r
        for k in range(2):
            @pl.when(r % 2 == k)
            def _unit(k=k):
                unit_compute(u, r, k, k)

        return carry

    lax.fori_loop(0, u_hi - u_lo, unit_body, 0)
    out_copy(u_lo, 0).wait()
    out_copy(u_lo, 1).wait()


@jax.jit
def _run(tab, raw):
    mesh = plsc.VectorSubcoreMesh(core_axis_name="c", subcore_axis_name="s")
    f = functools.partial(
        pl.kernel,
        mesh=mesh,
        compiler_params=pltpu.CompilerParams(needs_layout_passes=False),
        out_type=jax.ShapeDtypeStruct(
            (TIME_STEPS, NUM_SLOTS * EMB_DIM, BATCH), jnp.float32),
        scratch_types=[
            pltpu.VMEM((BATCH,), jnp.int32),
            pltpu.VMEM((BATCH,), jnp.int32),
            pltpu.VMEM((HALF, NUM_EMB), jnp.float32),
            pltpu.VMEM((HALF, NUM_EMB), jnp.float32),
            pltpu.VMEM((HALF, BATCH), jnp.float32),
            pltpu.VMEM((HALF, BATCH), jnp.float32),
            pltpu.SemaphoreType.DMA,
            pltpu.SemaphoreType.DMA,
            pltpu.SemaphoreType.DMA,
            pltpu.SemaphoreType.DMA,
        ],
    )(_sc_body)
    return f(tab, raw)


def kernel(sequence_bucket_inputs, tables):
    tab_t = tables.transpose(0, 1, 3, 2)
    raw_t = sequence_bucket_inputs.transpose(1, 2, 0)
    out = _run(tab_t, raw_t)
    return out.transpose(2, 0, 1)


# speedup vs baseline: 1.0084x; 1.0009x over previous
import functools

import jax
import jax.numpy as jnp
from jax import lax
from jax.experimental import pallas as pl
from jax.experimental.pallas import tpu as pltpu
from jax.experimental.pallas import tpu_sc as plsc

TIME_STEPS = 20
NUM_SLOTS = 18
NUM_EMB = 1002
EMB_DIM = 32
BATCH = 1024

NUM_WORKERS = 32
NUM_FACES = TIME_STEPS * NUM_SLOTS
LANES = 16
B_CHUNKS = BATCH // LANES
HALF = EMB_DIM // 2


NUM_UNITS = NUM_FACES * 2


def _sc_body(tab_hbm, raw_hbm, out_hbm,
             ids_a, ids_b, face_a, face_b, stage_a, stage_b,
             sem_ia, sem_ib, sem_oa, sem_ob):
    wid = lax.axis_index("s") * 2 + lax.axis_index("c")
    u_lo = (wid * NUM_UNITS) // NUM_WORKERS
    u_hi = ((wid + 1) * NUM_UNITS) // NUM_WORKERS
    idss, faces = (ids_a, ids_b), (face_a, face_b)
    sem_i = (sem_ia, sem_ib)
    stages, sem_o = (stage_a, stage_b), (sem_oa, sem_ob)

    def unit_tjsh(u):
        f = u // 2
        hh = u % 2
        j = f % NUM_SLOTS
        slot = j + 1 + (j >= 4).astype(jnp.int32)
        return f // NUM_SLOTS, j, slot, hh

    def in_copies(u, p):
        t, j, slot, hh = unit_tjsh(u)
        row = pl.multiple_of(HALF * hh, HALF)
        return (pltpu.make_async_copy(raw_hbm.at[t, slot], idss[p], sem_i[p]),
                pltpu.make_async_copy(
                    tab_hbm.at[t, j, pl.ds(row, HALF), :], faces[p],
                    sem_i[p]))

    def fire_in(u, p):
        for cp in in_copies(u, p):
            cp.start()

    def wait_in(u, p):
        for cp in in_copies(u, p):
            cp.wait()

    def out_copy(u, p):
        t, j, _, hh = unit_tjsh(u)
        row0 = pl.multiple_of(EMB_DIM * j + HALF * hh, HALF)
        return pltpu.make_async_copy(
            stages[p], out_hbm.at[t, pl.ds(row0, HALF), :], sem_o[p])

    def unit_compute(u, r, pi, ps):
        wait_in(u, pi)

        @pl.when(u + 1 < u_hi)
        def _prefetch():
            fire_in(u + 1, 1 - pi)

        @pl.when(r >= 2)
        def _drain():
            out_copy(u, ps).wait()

        def chunk_body(bc, carry, pi=pi, ps=ps):
            ev = idss[pi][pl.ds(bc * LANES, LANES)]
            vals = [
                plsc.load_gather(
                    faces[pi], [jnp.full((LANES,), d, jnp.int32), ev])
                for d in range(HALF)
            ]
            for d in range(HALF):
                stages[ps][d, pl.ds(bc * LANES, LANES)] = vals[d]
            return carry

        lax.fori_loop(0, B_CHUNKS, chunk_body, 0)
        out_copy(u, ps).start()

    fire_in(u_lo, 0)

    def unit_body(r, carry):
        u = u_lo + r
        for k in range(2):
            @pl.when(r % 2 == k)
            def _unit(k=k):
                unit_compute(u, r, k, k)

        return carry

    lax.fori_loop(0, u_hi - u_lo, unit_body, 0)
    out_copy(u_lo, 0).wait()
    out_copy(u_lo, 1).wait()


@jax.jit
def _run(tab, raw):
    mesh = plsc.VectorSubcoreMesh(core_axis_name="c", subcore_axis_name="s")
    f = functools.partial(
        pl.kernel,
        mesh=mesh,
        compiler_params=pltpu.CompilerParams(needs_layout_passes=False),
        out_type=jax.ShapeDtypeStruct(
            (TIME_STEPS, NUM_SLOTS * EMB_DIM, BATCH), jnp.float32),
        scratch_types=[
            pltpu.VMEM((BATCH,), jnp.int32),
            pltpu.VMEM((BATCH,), jnp.int32),
            pltpu.VMEM((HALF, NUM_EMB), jnp.float32),
            pltpu.VMEM((HALF, NUM_EMB), jnp.float32),
            pltpu.VMEM((HALF, BATCH), jnp.float32),
            pltpu.VMEM((HALF, BATCH), jnp.float32),
            pltpu.SemaphoreType.DMA,
            pltpu.SemaphoreType.DMA,
            pltpu.SemaphoreType.DMA,
            pltpu.SemaphoreType.DMA,
        ],
    )(_sc_body)
    return f(tab, raw)


def kernel(sequence_bucket_inputs, tables):
    tab_t = tables.transpose(0, 1, 3, 2)
    raw_t = sequence_bucket_inputs.transpose(1, 2, 0)
    out = _run(tab_t, raw_t)
    return out.transpose(2, 0, 1)
